# baseline (device time: 90506 ns/iter reference)
import functools

import jax
import jax.numpy as jnp
from jax import lax
from jax.experimental import pallas as pl
from jax.experimental.pallas import tpu as pltpu

N_DEV = 8
N_TOK = 2048
D_IN = 512
H_OUT = 1024
E_TOTAL = 64
E_LOCAL = E_TOTAL // N_DEV
CAP = 25
SLOTS = 32
ROWS = E_LOCAL * SLOTS
TOK_PER = N_TOK // N_DEV


def _moe_body(xg_ref, w_ref, p_ref, out_ref,
              sendbuf, recvbuf, send_sems, recv_sems):
    my_pos = lax.axis_index("i")

    barrier_sem = pltpu.get_barrier_semaphore()
    for j in range(N_DEV):
        @pl.when(j != my_pos)
        def _(j=j):
            pl.semaphore_signal(
                barrier_sem, inc=1,
                device_id=(j,), device_id_type=pl.DeviceIdType.MESH,
            )
    pl.semaphore_wait(barrier_sem, N_DEV - 1)

    ys = []
    for k in range(E_LOCAL):
        a = xg_ref[k * SLOTS:(k + 1) * SLOTS, :]
        w = w_ref[k]
        ys.append(jnp.dot(a, w, preferred_element_type=jnp.float32))
    ybuf = jnp.concatenate(ys, axis=0).astype(jnp.bfloat16)

    sendbuf[...] = jnp.dot(
        p_ref[...], ybuf, preferred_element_type=jnp.float32
    ).astype(jnp.bfloat16)

    recvbuf[pl.ds(my_pos * TOK_PER, TOK_PER), :] = (
        sendbuf[pl.ds(my_pos * TOK_PER, TOK_PER), :]
    )

    for j in range(N_DEV):
        @pl.when(j != my_pos)
        def _(j=j):
            rdma = pltpu.make_async_remote_copy(
                src_ref=sendbuf.at[pl.ds(j * TOK_PER, TOK_PER)],
                dst_ref=recvbuf.at[pl.ds(my_pos * TOK_PER, TOK_PER)],
                send_sem=send_sems.at[j],
                recv_sem=recv_sems.at[my_pos],
                device_id=(j,),
                device_id_type=pl.DeviceIdType.MESH,
            )
            rdma.start()

    for j in range(N_DEV):
        @pl.when(j != my_pos)
        def _(j=j):
            rdma = pltpu.make_async_remote_copy(
                src_ref=sendbuf.at[pl.ds(0, TOK_PER)],
                dst_ref=recvbuf.at[pl.ds(j * TOK_PER, TOK_PER)],
                send_sem=send_sems.at[j],
                recv_sem=recv_sems.at[j],
                device_id=(j,),
                device_id_type=pl.DeviceIdType.MESH,
            )
            rdma.wait_recv()

    total = recvbuf[0:TOK_PER, :].astype(jnp.float32)
    for d in range(1, N_DEV):
        total = total + recvbuf[d * TOK_PER:(d + 1) * TOK_PER, :].astype(
            jnp.float32
        )
    out_ref[...] = total

    for j in range(N_DEV):
        @pl.when(j != my_pos)
        def _(j=j):
            rdma = pltpu.make_async_remote_copy(
                src_ref=sendbuf.at[pl.ds(j * TOK_PER, TOK_PER)],
                dst_ref=recvbuf.at[pl.ds(0, TOK_PER)],
                send_sem=send_sems.at[j],
                recv_sem=recv_sems.at[0],
                device_id=(j,),
                device_id_type=pl.DeviceIdType.MESH,
            )
            rdma.wait_send()

    @functools.partial(pl.run_scoped, second_barrier=pltpu.SemaphoreType.REGULAR)
    def _(second_barrier):
        for j in range(N_DEV):
            @pl.when(j != my_pos)
            def _(j=j):
                pl.semaphore_signal(
                    second_barrier, inc=1,
                    device_id=(j,), device_id_type=pl.DeviceIdType.MESH,
                )
        pl.semaphore_wait(second_barrier, N_DEV - 1)


def kernel(x, router_W, route_idx, expert_W):
    my = lax.axis_index("i")
    e = route_idx[:, 0]
    oh = (e[:, None] == jnp.arange(E_TOTAL, dtype=e.dtype)[None, :]).astype(
        jnp.int32
    )
    pos = jnp.cumsum(oh, axis=0)
    slot = jnp.take_along_axis(pos, e[:, None].astype(jnp.int32), axis=1)[:, 0] - 1
    valid = slot < CAP
    s_c = jnp.where(valid, slot, SLOTS)

    slot_token = jnp.full((E_TOTAL, SLOTS), N_TOK, jnp.int32).at[e, s_c].set(
        jnp.arange(N_TOK, dtype=jnp.int32), mode="drop"
    )
    local_st = lax.dynamic_slice(slot_token, (my * E_LOCAL, 0), (E_LOCAL, SLOTS))
    dest = local_st.reshape(-1)

    x_pad = jnp.concatenate([x, jnp.zeros((1, D_IN), x.dtype)], axis=0)
    xg = x_pad[dest].astype(jnp.bfloat16)

    inv = jnp.full((N_TOK,), ROWS, jnp.int32).at[dest].set(
        jnp.arange(ROWS, dtype=jnp.int32), mode="drop"
    )
    P = (inv[:, None] == jnp.arange(ROWS, dtype=jnp.int32)[None, :]).astype(
        jnp.bfloat16
    )

    Wb = expert_W.astype(jnp.bfloat16)

    return pl.pallas_call(
        _moe_body,
        out_shape=jax.ShapeDtypeStruct((TOK_PER, H_OUT), jnp.float32),
        in_specs=[
            pl.BlockSpec(memory_space=pltpu.VMEM),
            pl.BlockSpec(memory_space=pltpu.VMEM),
            pl.BlockSpec(memory_space=pltpu.VMEM),
        ],
        out_specs=pl.BlockSpec(memory_space=pltpu.VMEM),
        scratch_shapes=[
            pltpu.VMEM((N_TOK, H_OUT), jnp.bfloat16),
            pltpu.VMEM((N_TOK, H_OUT), jnp.bfloat16),
            pltpu.SemaphoreType.DMA((N_DEV,)),
            pltpu.SemaphoreType.DMA((N_DEV,)),
        ],
        compiler_params=pltpu.CompilerParams(collective_id=0),
    )(xg, Wb, P)


# device time: 74317 ns/iter; 1.2178x vs baseline; 1.2178x over previous
import functools

import jax
import jax.numpy as jnp
from jax import lax
from jax.experimental import pallas as pl
from jax.experimental.pallas import tpu as pltpu

N_DEV = 8
N_TOK = 2048
D_IN = 512
H_OUT = 1024
E_TOTAL = 64
E_LOCAL = E_TOTAL // N_DEV
CAP = 25
SLOTS = 32
ROWS = E_LOCAL * SLOTS
TOK_PER = N_TOK // N_DEV


def _moe_body(e_ref, s_ref, x_ref, w_ref, out_ref,
              sendbuf, recvbuf, send_sems, recv_sems):
    my_pos = lax.axis_index("i")

    barrier_sem = pltpu.get_barrier_semaphore()
    for j in range(N_DEV):
        @pl.when(j != my_pos)
        def _(j=j):
            pl.semaphore_signal(
                barrier_sem, inc=1,
                device_id=(j,), device_id_type=pl.DeviceIdType.MESH,
            )
    pl.semaphore_wait(barrier_sem, N_DEV - 1)

    r_iota = lax.broadcasted_iota(jnp.int32, (ROWS, N_TOK), 0)
    k_of_r = r_iota // SLOTS
    c_of_r = r_iota % SLOTS
    e_row = e_ref[...]
    s_row = s_ref[...]
    G = (
        (e_row == my_pos * E_LOCAL + k_of_r)
        & (s_row == c_of_r)
        & (c_of_r < CAP)
    ).astype(jnp.bfloat16)

    xb = x_ref[...].astype(jnp.bfloat16)
    xg = jnp.dot(G, xb, preferred_element_type=jnp.float32).astype(
        jnp.bfloat16
    )

    ys = []
    for k in range(E_LOCAL):
        a = xg[k * SLOTS:(k + 1) * SLOTS, :]
        w = w_ref[k].astype(jnp.bfloat16)
        ys.append(jnp.dot(a, w, preferred_element_type=jnp.float32))
    ybuf = jnp.concatenate(ys, axis=0).astype(jnp.bfloat16)

    sendbuf[...] = lax.dot_general(
        G, ybuf, (((0,), (0,)), ((), ())),
        preferred_element_type=jnp.float32,
    ).astype(jnp.bfloat16)

    recvbuf[pl.ds(my_pos * TOK_PER, TOK_PER), :] = (
        sendbuf[pl.ds(my_pos * TOK_PER, TOK_PER), :]
    )

    for j in range(N_DEV):
        @pl.when(j != my_pos)
        def _(j=j):
            rdma = pltpu.make_async_remote_copy(
                src_ref=sendbuf.at[pl.ds(j * TOK_PER, TOK_PER)],
                dst_ref=recvbuf.at[pl.ds(my_pos * TOK_PER, TOK_PER)],
                send_sem=send_sems.at[j],
                recv_sem=recv_sems.at[my_pos],
                device_id=(j,),
                device_id_type=pl.DeviceIdType.MESH,
            )
            rdma.start()

    for j in range(N_DEV):
        @pl.when(j != my_pos)
        def _(j=j):
            rdma = pltpu.make_async_remote_copy(
                src_ref=sendbuf.at[pl.ds(0, TOK_PER)],
                dst_ref=recvbuf.at[pl.ds(j * TOK_PER, TOK_PER)],
                send_sem=send_sems.at[j],
                recv_sem=recv_sems.at[j],
                device_id=(j,),
                device_id_type=pl.DeviceIdType.MESH,
            )
            rdma.wait_recv()

    total = recvbuf[0:TOK_PER, :].astype(jnp.float32)
    for d in range(1, N_DEV):
        total = total + recvbuf[d * TOK_PER:(d + 1) * TOK_PER, :].astype(
            jnp.float32
        )
    out_ref[...] = total

    for j in range(N_DEV):
        @pl.when(j != my_pos)
        def _(j=j):
            rdma = pltpu.make_async_remote_copy(
                src_ref=sendbuf.at[pl.ds(j * TOK_PER, TOK_PER)],
                dst_ref=recvbuf.at[pl.ds(0, TOK_PER)],
                send_sem=send_sems.at[j],
                recv_sem=recv_sems.at[0],
                device_id=(j,),
                device_id_type=pl.DeviceIdType.MESH,
            )
            rdma.wait_send()

    @functools.partial(pl.run_scoped, second_barrier=pltpu.SemaphoreType.REGULAR)
    def _(second_barrier):
        for j in range(N_DEV):
            @pl.when(j != my_pos)
            def _(j=j):
                pl.semaphore_signal(
                    second_barrier, inc=1,
                    device_id=(j,), device_id_type=pl.DeviceIdType.MESH,
                )
        pl.semaphore_wait(second_barrier, N_DEV - 1)


def kernel(x, router_W, route_idx, expert_W):
    e = route_idx[:, 0].astype(jnp.int32)
    oh = (e[:, None] == jnp.arange(E_TOTAL, dtype=jnp.int32)[None, :]).astype(
        jnp.int32
    )
    pos = jnp.cumsum(oh, axis=0)
    slot = jnp.sum(pos * oh, axis=1) - 1

    e_row = e.reshape(1, N_TOK)
    s_row = slot.reshape(1, N_TOK).astype(jnp.int32)

    return pl.pallas_call(
        _moe_body,
        out_shape=jax.ShapeDtypeStruct((TOK_PER, H_OUT), jnp.float32),
        in_specs=[
            pl.BlockSpec(memory_space=pltpu.VMEM),
            pl.BlockSpec(memory_space=pltpu.VMEM),
            pl.BlockSpec(memory_space=pltpu.VMEM),
            pl.BlockSpec(memory_space=pltpu.VMEM),
        ],
        out_specs=pl.BlockSpec(memory_space=pltpu.VMEM),
        scratch_shapes=[
            pltpu.VMEM((N_TOK, H_OUT), jnp.bfloat16),
            pltpu.VMEM((N_TOK, H_OUT), jnp.bfloat16),
            pltpu.SemaphoreType.DMA((N_DEV,)),
            pltpu.SemaphoreType.DMA((N_DEV,)),
        ],
        compiler_params=pltpu.CompilerParams(collective_id=0),
    )(e_row, s_row, x, expert_W)


# device time: 58913 ns/iter; 1.5363x vs baseline; 1.2615x over previous
import functools

import jax
import jax.numpy as jnp
from jax import lax
from jax.experimental import pallas as pl
from jax.experimental.pallas import tpu as pltpu

N_DEV = 8
N_TOK = 2048
D_IN = 512
H_OUT = 1024
E_TOTAL = 64
E_LOCAL = E_TOTAL // N_DEV
CAP = 25
SLOTS = 32
ROWS = E_LOCAL * SLOTS
TOK_PER = N_TOK // N_DEV


def _moe_body(ri_ref, x_ref, w_ref, out_ref,
              sendbuf, recvbuf, send_sems, recv_sems):
    my_pos = lax.axis_index("i")

    barrier_sem = pltpu.get_barrier_semaphore()
    for j in range(N_DEV):
        @pl.when(j != my_pos)
        def _(j=j):
            pl.semaphore_signal(
                barrier_sem, inc=1,
                device_id=(j,), device_id_type=pl.DeviceIdType.MESH,
            )
    pl.semaphore_wait(barrier_sem, N_DEV - 1)

    e_col = ri_ref[...]
    oh = (e_col == lax.broadcasted_iota(jnp.int32, (N_TOK, E_TOTAL), 1)).astype(
        jnp.bfloat16
    )
    L = (
        lax.broadcasted_iota(jnp.int32, (N_TOK, N_TOK), 0)
        >= lax.broadcasted_iota(jnp.int32, (N_TOK, N_TOK), 1)
    ).astype(jnp.bfloat16)
    pos = jnp.dot(L, oh, preferred_element_type=jnp.float32)
    s_col = (
        jnp.sum(pos * oh.astype(jnp.float32), axis=1, keepdims=True) - 1.0
    ).astype(jnp.int32)

    r_iota = lax.broadcasted_iota(jnp.int32, (N_TOK, ROWS), 1)
    k_of_r = r_iota // SLOTS
    c_of_r = r_iota % SLOTS
    Gt = (
        (e_col == my_pos * E_LOCAL + k_of_r)
        & (s_col == c_of_r)
        & (c_of_r < CAP)
    ).astype(jnp.bfloat16)

    xb = x_ref[...].astype(jnp.bfloat16)
    xg = lax.dot_general(
        Gt, xb, (((0,), (0,)), ((), ())),
        preferred_element_type=jnp.float32,
    ).astype(jnp.bfloat16)

    ys = []
    for k in range(E_LOCAL):
        a = xg[k * SLOTS:(k + 1) * SLOTS, :]
        w = w_ref[k].astype(jnp.bfloat16)
        ys.append(jnp.dot(a, w, preferred_element_type=jnp.float32))
    ybuf = jnp.concatenate(ys, axis=0).astype(jnp.bfloat16)

    for j in range(N_DEV):
        blk = slice(j * TOK_PER, (j + 1) * TOK_PER)
        sendbuf[blk, :] = jnp.dot(
            Gt[blk, :], ybuf, preferred_element_type=jnp.float32
        ).astype(jnp.bfloat16)

        @pl.when(j == my_pos)
        def _(j=j, blk=blk):
            recvbuf[pl.ds(my_pos * TOK_PER, TOK_PER), :] = sendbuf[blk, :]

        @pl.when(j != my_pos)
        def _(j=j):
            rdma = pltpu.make_async_remote_copy(
                src_ref=sendbuf.at[pl.ds(j * TOK_PER, TOK_PER)],
                dst_ref=recvbuf.at[pl.ds(my_pos * TOK_PER, TOK_PER)],
                send_sem=send_sems.at[j],
                recv_sem=recv_sems.at[my_pos],
                device_id=(j,),
                device_id_type=pl.DeviceIdType.MESH,
            )
            rdma.start()

    for j in range(N_DEV):
        @pl.when(j != my_pos)
        def _(j=j):
            rdma = pltpu.make_async_remote_copy(
                src_ref=sendbuf.at[pl.ds(0, TOK_PER)],
                dst_ref=recvbuf.at[pl.ds(j * TOK_PER, TOK_PER)],
                send_sem=send_sems.at[j],
                recv_sem=recv_sems.at[j],
                device_id=(j,),
                device_id_type=pl.DeviceIdType.MESH,
            )
            rdma.wait_recv()

    total = recvbuf[0:TOK_PER, :].astype(jnp.float32)
    for d in range(1, N_DEV):
        total = total + recvbuf[d * TOK_PER:(d + 1) * TOK_PER, :].astype(
            jnp.float32
        )
    out_ref[...] = total

    for j in range(N_DEV):
        @pl.when(j != my_pos)
        def _(j=j):
            rdma = pltpu.make_async_remote_copy(
                src_ref=sendbuf.at[pl.ds(j * TOK_PER, TOK_PER)],
                dst_ref=recvbuf.at[pl.ds(0, TOK_PER)],
                send_sem=send_sems.at[j],
                recv_sem=recv_sems.at[0],
                device_id=(j,),
                device_id_type=pl.DeviceIdType.MESH,
            )
            rdma.wait_send()

    @functools.partial(pl.run_scoped, second_barrier=pltpu.SemaphoreType.REGULAR)
    def _(second_barrier):
        for j in range(N_DEV):
            @pl.when(j != my_pos)
            def _(j=j):
                pl.semaphore_signal(
                    second_barrier, inc=1,
                    device_id=(j,), device_id_type=pl.DeviceIdType.MESH,
                )
        pl.semaphore_wait(second_barrier, N_DEV - 1)


def kernel(x, router_W, route_idx, expert_W):
    return pl.pallas_call(
        _moe_body,
        out_shape=jax.ShapeDtypeStruct((TOK_PER, H_OUT), jnp.float32),
        in_specs=[
            pl.BlockSpec(memory_space=pltpu.VMEM),
            pl.BlockSpec(memory_space=pltpu.VMEM),
            pl.BlockSpec(memory_space=pltpu.VMEM),
        ],
        out_specs=pl.BlockSpec(memory_space=pltpu.VMEM),
        scratch_shapes=[
            pltpu.VMEM((N_TOK, H_OUT), jnp.bfloat16),
            pltpu.VMEM((N_TOK, H_OUT), jnp.bfloat16),
            pltpu.SemaphoreType.DMA((N_DEV,)),
            pltpu.SemaphoreType.DMA((N_DEV,)),
        ],
        compiler_params=pltpu.CompilerParams(collective_id=0),
    )(route_idx.astype(jnp.int32), x, expert_W)


# device time: 52894 ns/iter; 1.7111x vs baseline; 1.1138x over previous
import functools

import jax
import jax.numpy as jnp
from jax import lax
from jax.experimental import pallas as pl
from jax.experimental.pallas import tpu as pltpu

N_DEV = 8
N_TOK = 2048
D_IN = 512
H_OUT = 1024
E_TOTAL = 64
E_LOCAL = E_TOTAL // N_DEV
CAP = 25
SLOTS = 32
ROWS = E_LOCAL * SLOTS
TOK_PER = N_TOK // N_DEV


def _moe_body(ri_ref, x_hbm, w_hbm, out_ref,
              xv, wv, sendbuf, recvbuf,
              x_dma_sem, w_dma_sem, send_sems, recv_sems):
    my_pos = lax.axis_index("i")

    x_dma = pltpu.make_async_copy(x_hbm, xv, x_dma_sem)
    x_dma.start()
    w_dma = pltpu.make_async_copy(w_hbm, wv, w_dma_sem)
    w_dma.start()

    barrier_sem = pltpu.get_barrier_semaphore()
    for j in range(N_DEV):
        @pl.when(j != my_pos)
        def _(j=j):
            pl.semaphore_signal(
                barrier_sem, inc=1,
                device_id=(j,), device_id_type=pl.DeviceIdType.MESH,
            )
    pl.semaphore_wait(barrier_sem, N_DEV - 1)

    e_col = ri_ref[...]
    oh = (e_col == lax.broadcasted_iota(jnp.int32, (N_TOK, E_TOTAL), 1)).astype(
        jnp.bfloat16
    )
    L = (
        lax.broadcasted_iota(jnp.int32, (N_TOK, N_TOK), 0)
        >= lax.broadcasted_iota(jnp.int32, (N_TOK, N_TOK), 1)
    ).astype(jnp.bfloat16)
    pos = jnp.dot(L, oh, preferred_element_type=jnp.float32)
    s_col = (
        jnp.sum(pos * oh.astype(jnp.float32), axis=1, keepdims=True) - 1.0
    ).astype(jnp.int32)

    r_iota = lax.broadcasted_iota(jnp.int32, (N_TOK, ROWS), 1)
    k_of_r = r_iota // SLOTS
    c_of_r = r_iota % SLOTS
    Gt = (
        (e_col == my_pos * E_LOCAL + k_of_r)
        & (s_col == c_of_r)
        & (c_of_r < CAP)
    ).astype(jnp.bfloat16)

    x_dma.wait()
    xb = xv[...].astype(jnp.bfloat16)
    xg = lax.dot_general(
        Gt, xb, (((0,), (0,)), ((), ())),
        preferred_element_type=jnp.float32,
    ).astype(jnp.bfloat16)

    w_dma.wait()
    ys = []
    for k in range(E_LOCAL):
        a = xg[k * SLOTS:(k + 1) * SLOTS, :]
        w = wv[k].astype(jnp.bfloat16)
        ys.append(jnp.dot(a, w, preferred_element_type=jnp.float32))
    ybuf = jnp.concatenate(ys, axis=0).astype(jnp.bfloat16)

    sendbuf[...] = jnp.dot(
        Gt, ybuf, preferred_element_type=jnp.float32
    ).astype(jnp.bfloat16)

    for o in range(1, N_DEV):
        j = (my_pos + o) % N_DEV
        rdma = pltpu.make_async_remote_copy(
            src_ref=sendbuf.at[pl.ds(j * TOK_PER, TOK_PER)],
            dst_ref=recvbuf.at[pl.ds(my_pos * TOK_PER, TOK_PER)],
            send_sem=send_sems.at[j],
            recv_sem=recv_sems.at[my_pos],
            device_id=(j,),
            device_id_type=pl.DeviceIdType.MESH,
        )
        rdma.start()

    total = sendbuf[pl.ds(my_pos * TOK_PER, TOK_PER), :].astype(jnp.float32)

    for o in range(1, N_DEV):
        j = (my_pos + N_DEV - o) % N_DEV
        rdma = pltpu.make_async_remote_copy(
            src_ref=sendbuf.at[pl.ds(0, TOK_PER)],
            dst_ref=recvbuf.at[pl.ds(j * TOK_PER, TOK_PER)],
            send_sem=send_sems.at[0],
            recv_sem=recv_sems.at[j],
            device_id=(j,),
            device_id_type=pl.DeviceIdType.MESH,
        )
        rdma.wait_recv()
        total = total + recvbuf[pl.ds(j * TOK_PER, TOK_PER), :].astype(
            jnp.float32
        )
    out_ref[...] = total

    for o in range(1, N_DEV):
        j = (my_pos + o) % N_DEV
        rdma = pltpu.make_async_remote_copy(
            src_ref=sendbuf.at[pl.ds(j * TOK_PER, TOK_PER)],
            dst_ref=recvbuf.at[pl.ds(0, TOK_PER)],
            send_sem=send_sems.at[j],
            recv_sem=recv_sems.at[0],
            device_id=(j,),
            device_id_type=pl.DeviceIdType.MESH,
        )
        rdma.wait_send()

    @functools.partial(pl.run_scoped, second_barrier=pltpu.SemaphoreType.REGULAR)
    def _(second_barrier):
        for j in range(N_DEV):
            @pl.when(j != my_pos)
            def _(j=j):
                pl.semaphore_signal(
                    second_barrier, inc=1,
                    device_id=(j,), device_id_type=pl.DeviceIdType.MESH,
                )
        pl.semaphore_wait(second_barrier, N_DEV - 1)


def kernel(x, router_W, route_idx, expert_W):
    return pl.pallas_call(
        _moe_body,
        out_shape=jax.ShapeDtypeStruct((TOK_PER, H_OUT), jnp.float32),
        in_specs=[
            pl.BlockSpec(memory_space=pltpu.VMEM),
            pl.BlockSpec(memory_space=pl.ANY),
            pl.BlockSpec(memory_space=pl.ANY),
        ],
        out_specs=pl.BlockSpec(memory_space=pltpu.VMEM),
        scratch_shapes=[
            pltpu.VMEM((N_TOK, D_IN), jnp.float32),
            pltpu.VMEM((E_LOCAL, D_IN, H_OUT), jnp.float32),
            pltpu.VMEM((N_TOK, H_OUT), jnp.bfloat16),
            pltpu.VMEM((N_TOK, H_OUT), jnp.bfloat16),
            pltpu.SemaphoreType.DMA,
            pltpu.SemaphoreType.DMA,
            pltpu.SemaphoreType.DMA((N_DEV,)),
            pltpu.SemaphoreType.DMA((N_DEV,)),
        ],
        compiler_params=pltpu.CompilerParams(collective_id=0),
    )(route_idx.astype(jnp.int32), x, expert_W)


# device time: 21929 ns/iter; 4.1272x vs baseline; 2.4121x over previous
import functools
import os

import jax
import jax.numpy as jnp
from jax import lax
from jax.experimental import pallas as pl
from jax.experimental.pallas import tpu as pltpu

_ABLATE = os.environ.get("ABLATE", "")

N_DEV = 8
N_TOK = 2048
D_IN = 512
H_OUT = 1024
E_TOTAL = 64
E_LOCAL = E_TOTAL // N_DEV
CAP = 25
SLOTS = 32
ROWS = E_LOCAL * SLOTS
TOK_PER = N_TOK // N_DEV


def _moe_body(ri_ref, x_hbm, w_hbm, out_ref,
              xv, wv, sendbuf, recvbuf,
              x_dma_sem, w_dma_sem, send_sems, recv_sems):
    my_pos = lax.axis_index("i")

    if _ABLATE != "nocompute":
        x_dma = pltpu.make_async_copy(x_hbm, xv, x_dma_sem)
        x_dma.start()
        w_dma = pltpu.make_async_copy(w_hbm, wv, w_dma_sem)
        w_dma.start()

    if _ABLATE != "nocomm":
        barrier_sem = pltpu.get_barrier_semaphore()
        for j in range(N_DEV):
            @pl.when(j != my_pos)
            def _(j=j):
                pl.semaphore_signal(
                    barrier_sem, inc=1,
                    device_id=(j,), device_id_type=pl.DeviceIdType.MESH,
                )
        pl.semaphore_wait(barrier_sem, N_DEV - 1)

    if _ABLATE == "nocompute":
        sendbuf[...] = jnp.zeros((N_TOK, H_OUT), jnp.bfloat16)
        _comm(my_pos, out_ref, sendbuf, recvbuf, send_sems, recv_sems)
        return

    e_col = ri_ref[...]
    oh = (e_col == lax.broadcasted_iota(jnp.int32, (N_TOK, E_TOTAL), 1)).astype(
        jnp.bfloat16
    )
    L = (
        lax.broadcasted_iota(jnp.int32, (N_TOK, N_TOK), 0)
        >= lax.broadcasted_iota(jnp.int32, (N_TOK, N_TOK), 1)
    ).astype(jnp.bfloat16)
    pos = jnp.dot(L, oh, preferred_element_type=jnp.float32)
    s_col = (
        jnp.sum(pos * oh.astype(jnp.float32), axis=1, keepdims=True) - 1.0
    ).astype(jnp.int32)

    r_iota = lax.broadcasted_iota(jnp.int32, (N_TOK, ROWS), 1)
    k_of_r = r_iota // SLOTS
    c_of_r = r_iota % SLOTS
    Gt = (
        (e_col == my_pos * E_LOCAL + k_of_r)
        & (s_col == c_of_r)
        & (c_of_r < CAP)
    ).astype(jnp.bfloat16)

    x_dma.wait()
    xb = xv[...].astype(jnp.bfloat16)
    xg = lax.dot_general(
        Gt, xb, (((0,), (0,)), ((), ())),
        preferred_element_type=jnp.float32,
    ).astype(jnp.bfloat16)

    w_dma.wait()
    ys = []
    for k in range(E_LOCAL):
        a = xg[k * SLOTS:(k + 1) * SLOTS, :]
        w = wv[k].astype(jnp.bfloat16)
        ys.append(jnp.dot(a, w, preferred_element_type=jnp.float32))
    ybuf = jnp.concatenate(ys, axis=0).astype(jnp.bfloat16)

    sendbuf[...] = jnp.dot(
        Gt, ybuf, preferred_element_type=jnp.float32
    ).astype(jnp.bfloat16)

    if _ABLATE == "nocomm":
        out_ref[...] = sendbuf[pl.ds(my_pos * TOK_PER, TOK_PER), :].astype(
            jnp.float32
        )
        return
    _comm(my_pos, out_ref, sendbuf, recvbuf, send_sems, recv_sems)


def _comm(my_pos, out_ref, sendbuf, recvbuf, send_sems, recv_sems):
    for o in range(1, N_DEV):
        j = (my_pos + o) % N_DEV
        rdma = pltpu.make_async_remote_copy(
            src_ref=sendbuf.at[pl.ds(j * TOK_PER, TOK_PER)],
            dst_ref=recvbuf.at[pl.ds(my_pos * TOK_PER, TOK_PER)],
            send_sem=send_sems.at[j],
            recv_sem=recv_sems.at[my_pos],
            device_id=(j,),
            device_id_type=pl.DeviceIdType.MESH,
        )
        rdma.start()

    total = sendbuf[pl.ds(my_pos * TOK_PER, TOK_PER), :].astype(jnp.float32)

    for o in range(1, N_DEV):
        j = (my_pos + N_DEV - o) % N_DEV
        rdma = pltpu.make_async_remote_copy(
            src_ref=sendbuf.at[pl.ds(0, TOK_PER)],
            dst_ref=recvbuf.at[pl.ds(j * TOK_PER, TOK_PER)],
            send_sem=send_sems.at[0],
            recv_sem=recv_sems.at[j],
            device_id=(j,),
            device_id_type=pl.DeviceIdType.MESH,
        )
        rdma.wait_recv()
        total = total + recvbuf[pl.ds(j * TOK_PER, TOK_PER), :].astype(
            jnp.float32
        )
    out_ref[...] = total

    for o in range(1, N_DEV):
        j = (my_pos + o) % N_DEV
        rdma = pltpu.make_async_remote_copy(
            src_ref=sendbuf.at[pl.ds(j * TOK_PER, TOK_PER)],
            dst_ref=recvbuf.at[pl.ds(0, TOK_PER)],
            send_sem=send_sems.at[j],
            recv_sem=recv_sems.at[0],
            device_id=(j,),
            device_id_type=pl.DeviceIdType.MESH,
        )
        rdma.wait_send()

    @functools.partial(pl.run_scoped, second_barrier=pltpu.SemaphoreType.REGULAR)
    def _(second_barrier):
        for j in range(N_DEV):
            @pl.when(j != my_pos)
            def _(j=j):
                pl.semaphore_signal(
                    second_barrier, inc=1,
                    device_id=(j,), device_id_type=pl.DeviceIdType.MESH,
                )
        pl.semaphore_wait(second_barrier, N_DEV - 1)


def kernel(x, router_W, route_idx, expert_W):
    return pl.pallas_call(
        _moe_body,
        out_shape=jax.ShapeDtypeStruct((TOK_PER, H_OUT), jnp.float32),
        in_specs=[
            pl.BlockSpec(memory_space=pltpu.VMEM),
            pl.BlockSpec(memory_space=pl.ANY),
            pl.BlockSpec(memory_space=pl.ANY),
        ],
        out_specs=pl.BlockSpec(memory_space=pltpu.VMEM),
        scratch_shapes=[
            pltpu.VMEM((N_TOK, D_IN), jnp.float32),
            pltpu.VMEM((E_LOCAL, D_IN, H_OUT), jnp.float32),
            pltpu.VMEM((N_TOK, H_OUT), jnp.bfloat16),
            pltpu.VMEM((N_TOK, H_OUT), jnp.bfloat16),
            pltpu.SemaphoreType.DMA,
            pltpu.SemaphoreType.DMA,
            pltpu.SemaphoreType.DMA((N_DEV,)),
            pltpu.SemaphoreType.DMA((N_DEV,)),
        ],
        compiler_params=pltpu.CompilerParams(
            collective_id=None if _ABLATE == "nocomm" else 0
        ),
    )(route_idx.astype(jnp.int32), x, expert_W)
